# Initial kernel scaffold; baseline (speedup 1.0000x reference)
import jax
import jax.numpy as jnp
from jax.experimental import pallas as pl


def kernel(correlation_tensor, src_lbl_batch_resize):
    raise NotImplementedError("write your pallas kernel here")


# speedup vs baseline: 265.7458x; 265.7458x over previous
import functools

import jax
import jax.numpy as jnp
from jax import lax
from jax.experimental import pallas as pl

BS = 2
S = 2304
T = 2304
K = 20
NC = 21
TT = 768


def _topk_combine_kernel(corr_ref, lbl_ref, out_ref):
    c = corr_ref[0]
    m = jnp.max(c, axis=0)

    def body(_, m):
        nxt = jnp.where(c < m[None, :], c, -jnp.inf)
        return jnp.max(nxt, axis=0)

    th = lax.fori_loop(0, K - 1, body, m)
    masked = jnp.where(c >= th[None, :], c, 0.0)
    out_ref[0] = lax.dot_general(
        lbl_ref[0], masked,
        dimension_numbers=(((1,), (0,)), ((), ())),
        preferred_element_type=jnp.float32,
    )


@jax.jit
def kernel(correlation_tensor, src_lbl_batch_resize):
    bs, _, h, w = correlation_tensor.shape
    corr = correlation_tensor.reshape(bs, S, T)
    lbl = src_lbl_batch_resize.reshape(bs, NC, S)

    out = pl.pallas_call(
        _topk_combine_kernel,
        grid=(bs, T // TT),
        in_specs=[
            pl.BlockSpec((1, S, TT), lambda b, t: (b, 0, t)),
            pl.BlockSpec((1, NC, S), lambda b, t: (b, 0, 0)),
        ],
        out_specs=pl.BlockSpec((1, NC, TT), lambda b, t: (b, 0, t)),
        out_shape=jax.ShapeDtypeStruct((bs, NC, T), jnp.float32),
    )(corr, lbl)
    return out.reshape(bs, NC, h, w)
